# baseline (speedup 1.0000x reference)
import functools

import jax
import jax.numpy as jnp
from jax import lax
from jax.experimental import pallas as pl
from jax.experimental.pallas import tpu as pltpu
from jax.experimental.pallas import tpu_sc as plsc

N = 32768
B = 16
D = 128
L = 16
DC = D // L
NC, NS = 2, 16
NW = NC * NS
RPW = N // NW
CH = 256
G = RPW // CH
KI = 128
_MESH = plsc.VectorSubcoreMesh(core_axis_name="c", subcore_axis_name="s")


@functools.partial(
    pl.kernel,
    out_type=jax.ShapeDtypeStruct((NC, B, D), jnp.float32),
    mesh=_MESH,
    scratch_types=[
        pltpu.VMEM((RPW // KI, KI), jnp.int32),
        pltpu.VMEM((2, CH, D), jnp.float32),
        pltpu.VMEM((B, D), jnp.float32),
        pltpu.VMEM_SHARED((B, D), jnp.float32),
        pltpu.SemaphoreType.DMA,
        pltpu.SemaphoreType.DMA,
    ],
)
def _seg_sum_sc(h_hbm, batch_hbm, out_hbm, bidx, buf, zbuf, acc_sh,
                sem0, sem1):
    cid = lax.axis_index("c")
    sid = lax.axis_index("s")
    wid = sid * NC + cid
    row0 = wid * RPW
    pltpu.sync_copy(batch_hbm.at[pl.ds(wid * (RPW // KI), RPW // KI)], bidx)

    @pl.when(sid == 0)
    def _():
        zv = jnp.zeros((L,), jnp.float32)
        for i in range(B):
            for c in range(DC):
                zbuf[i, pl.ds(c * L, L)] = zv
        pltpu.sync_copy(zbuf, acc_sh)

    plsc.subcore_barrier()
    sems = (sem0, sem1)
    cps = [pltpu.async_copy(h_hbm.at[pl.ds(row0, CH)], buf.at[0], sems[0]),
           None]
    for g in range(G):
        b = g % 2
        if g + 1 < G:
            cps[1 - b] = pltpu.async_copy(
                h_hbm.at[pl.ds(row0 + (g + 1) * CH, CH)], buf.at[1 - b],
                sems[1 - b])
        cps[b].wait()
        for k in range(CH // KI):
            pltpu.sync_copy(buf.at[b, pl.ds(k * KI, KI)],
                            acc_sh.at[bidx.at[g * (CH // KI) + k]],
                            add=True)
    plsc.subcore_barrier()

    @pl.when(sid == 0)
    def _():
        pltpu.sync_copy(acc_sh, out_hbm.at[cid])


@functools.partial(
    pl.kernel,
    out_type=jax.ShapeDtypeStruct((N * D,), jnp.float32),
    mesh=_MESH,
    scratch_types=[
        pltpu.VMEM((RPW,), jnp.int32),
        pltpu.VMEM((2, CH * D), jnp.float32),
        pltpu.VMEM((B * D,), jnp.float32),
        pltpu.SemaphoreType.DMA,
        pltpu.SemaphoreType.DMA,
    ],
)
def _gather_add_sc(h_hbm, batch_hbm, vn_hbm, out_hbm, batch_v, buf, vn_v,
                   sem0, sem1):
    wid = lax.axis_index("s") * NC + lax.axis_index("c")
    row0 = wid * RPW
    pltpu.sync_copy(batch_hbm.at[pl.ds(row0, RPW)], batch_v)
    pltpu.sync_copy(vn_hbm, vn_v)
    sems = (sem0, sem1)
    cps = [pltpu.async_copy(h_hbm.at[pl.ds(row0 * D, CH * D)], buf.at[0],
                            sems[0]), None]
    for g in range(G):
        b = g % 2
        if g + 1 < G:
            cps[1 - b] = pltpu.async_copy(
                h_hbm.at[pl.ds((row0 + (g + 1) * CH) * D, CH * D)],
                buf.at[1 - b], sems[1 - b])
        cps[b].wait()
        s_first = batch_v[pl.ds(g * CH, L)][0]
        s_last = batch_v[pl.ds(g * CH + CH - L, L)][L - 1]

        def fast(g=g, b=b, s=s_first):
            vnrow = [vn_v[pl.ds(s * D + c * L, L)] for c in range(DC)]

            @plsc.parallel_loop(0, CH // L, 1, unroll=2)
            def _(t):
                for j in range(L):
                    for c in range(DC):
                        sl = pl.ds(t * (L * D) + j * D + c * L, L)
                        buf[b, sl] = buf[b, sl] + vnrow[c]

        def slow(g=g, b=b):
            @plsc.parallel_loop(0, CH // L, 1)
            def _(t):
                segv = batch_v[pl.ds(g * CH + t * L, L)]
                for j in range(L):
                    o = segv[j] * D
                    for c in range(DC):
                        plsc.addupdate(
                            buf.at[b, pl.ds(t * (L * D) + j * D + c * L, L)],
                            vn_v[pl.ds(o + c * L, L)])

        lax.cond(s_first == s_last, fast, slow)

        pltpu.sync_copy(buf.at[b],
                        out_hbm.at[pl.ds((row0 + g * CH) * D, CH * D)])


def _mlp_body(p_ref, vn_ref, w1_ref, w2_ref, o_ref):
    x = jnp.sum(p_ref[...], axis=0) + vn_ref[...]
    y = jnp.maximum(
        lax.dot_general(x, w1_ref[...], (((1,), (1,)), ((), ())),
                        preferred_element_type=jnp.float32), 0.0)
    o_ref[...] = lax.dot_general(y, w2_ref[...], (((1,), (1,)), ((), ())),
                                 preferred_element_type=jnp.float32)


_mlp_tc = pl.pallas_call(
    _mlp_body,
    out_shape=jax.ShapeDtypeStruct((B, D), jnp.float32),
)


def kernel(h, batch, vn_h, W1, W2):
    batch = batch.astype(jnp.int32)
    partial = _seg_sum_sc(h, batch.reshape(N // 128, 128))
    vn_new = _mlp_tc(partial, vn_h, W1, W2)
    h_new = _gather_add_sc(h.reshape(N * D), batch, vn_new.reshape(B * D))
    return (h_new.reshape(N, D), vn_new)


# speedup vs baseline: 1.1076x; 1.1076x over previous
import functools

import jax
import jax.numpy as jnp
from jax import lax
from jax.experimental import pallas as pl
from jax.experimental.pallas import tpu as pltpu
from jax.experimental.pallas import tpu_sc as plsc

N = 32768
B = 16
D = 128
L = 16
DC = D // L
NC, NS = 2, 16
NW = NC * NS
RPW = N // NW
CH = 256
G = RPW // CH
KI = 128
_MESH = plsc.VectorSubcoreMesh(core_axis_name="c", subcore_axis_name="s")


@functools.partial(
    pl.kernel,
    out_type=jax.ShapeDtypeStruct((NC, B, D), jnp.float32),
    mesh=_MESH,
    scratch_types=[
        pltpu.VMEM((RPW // KI, KI), jnp.int32),
        pltpu.VMEM((2, CH, D), jnp.float32),
        pltpu.VMEM((B, D), jnp.float32),
        pltpu.VMEM_SHARED((B, D), jnp.float32),
        pltpu.SemaphoreType.DMA,
        pltpu.SemaphoreType.DMA,
    ],
)
def _seg_sum_sc(h_hbm, batch_hbm, out_hbm, bidx, buf, zbuf, acc_sh,
                sem0, sem1):
    cid = lax.axis_index("c")
    sid = lax.axis_index("s")
    wid = sid * NC + cid
    row0 = wid * RPW
    pltpu.sync_copy(batch_hbm.at[pl.ds(wid * (RPW // KI), RPW // KI)], bidx)

    @pl.when(sid == 0)
    def _():
        zv = jnp.zeros((L,), jnp.float32)
        for i in range(B):
            for c in range(DC):
                zbuf[i, pl.ds(c * L, L)] = zv
        pltpu.sync_copy(zbuf, acc_sh)

    plsc.subcore_barrier()
    sems = (sem0, sem1)
    cps = [pltpu.async_copy(h_hbm.at[pl.ds(row0, CH)], buf.at[0], sems[0]),
           None]
    for g in range(G):
        b = g % 2
        if g + 1 < G:
            cps[1 - b] = pltpu.async_copy(
                h_hbm.at[pl.ds(row0 + (g + 1) * CH, CH)], buf.at[1 - b],
                sems[1 - b])
        cps[b].wait()
        for k in range(CH // KI):
            pltpu.sync_copy(buf.at[b, pl.ds(k * KI, KI)],
                            acc_sh.at[bidx.at[g * (CH // KI) + k]],
                            add=True)
    plsc.subcore_barrier()

    @pl.when(sid == 0)
    def _():
        pltpu.sync_copy(acc_sh, out_hbm.at[cid])


@functools.partial(
    pl.kernel,
    out_type=jax.ShapeDtypeStruct((N * D,), jnp.float32),
    mesh=_MESH,
    scratch_types=[
        pltpu.VMEM((RPW,), jnp.int32),
        (pltpu.VMEM((CH * D,), jnp.float32),
         pltpu.VMEM((CH * D,), jnp.float32),
         pltpu.VMEM((CH * D,), jnp.float32)),
        pltpu.VMEM((B * D,), jnp.float32),
        (pltpu.SemaphoreType.DMA, pltpu.SemaphoreType.DMA,
         pltpu.SemaphoreType.DMA),
        (pltpu.SemaphoreType.DMA, pltpu.SemaphoreType.DMA,
         pltpu.SemaphoreType.DMA),
    ],
)
def _gather_add_sc(h_hbm, batch_hbm, vn_hbm, out_hbm, batch_v, buf, vn_v,
                   isems, osems):
    wid = lax.axis_index("s") * NC + lax.axis_index("c")
    row0 = wid * RPW
    pltpu.sync_copy(batch_hbm.at[pl.ds(row0, RPW)], batch_v)
    pltpu.sync_copy(vn_hbm, vn_v)
    icps = [None, None, None]
    ocps = [None, None, None]
    for m in range(min(2, G)):
        icps[m] = pltpu.async_copy(
            h_hbm.at[pl.ds((row0 + m * CH) * D, CH * D)], buf[m],
            isems[m])
    for g in range(G):
        m = g % 3
        if g >= 2:
            ocps[(g - 2) % 3].wait()
        if g + 1 < G and g >= 1:
            icps[(g + 1) % 3] = pltpu.async_copy(
                h_hbm.at[pl.ds((row0 + (g + 1) * CH) * D, CH * D)],
                buf[(g + 1) % 3], isems[(g + 1) % 3])
        icps[m].wait()
        s_first = batch_v[pl.ds(g * CH, L)][0]
        s_last = batch_v[pl.ds(g * CH + CH - L, L)][L - 1]

        def fast(g=g, m=m, s=s_first):
            vnrow = [vn_v[pl.ds(s * D + c * L, L)] for c in range(DC)]

            def fgrp(t, carry):
                for j in range(L):
                    for c in range(DC):
                        plsc.addupdate(
                            buf[m].at[pl.ds(t * (L * D) + j * D + c * L, L)],
                            vnrow[c])
                return carry

            lax.fori_loop(0, CH // L, fgrp, 0)

        def slow(g=g, m=m):
            def grp_body(t, carry):
                segv = batch_v[pl.ds(g * CH + t * L, L)]
                for j in range(L):
                    o = segv[j] * D
                    for c in range(DC):
                        plsc.addupdate(
                            buf[m].at[pl.ds(t * (L * D) + j * D + c * L, L)],
                            vn_v[pl.ds(o + c * L, L)])
                return carry

            lax.fori_loop(0, CH // L, grp_body, 0)

        lax.cond(s_first == s_last, fast, slow)

        ocps[m] = pltpu.async_copy(
            buf[m], out_hbm.at[pl.ds((row0 + g * CH) * D, CH * D)],
            osems[m])
    for g in range(max(0, G - 2), G):
        ocps[g % 3].wait()


def _mlp_body(p_ref, vn_ref, w1_ref, w2_ref, o_ref):
    x = jnp.sum(p_ref[...], axis=0) + vn_ref[...]
    y = jnp.maximum(
        lax.dot_general(x, w1_ref[...], (((1,), (1,)), ((), ())),
                        preferred_element_type=jnp.float32), 0.0)
    o_ref[...] = lax.dot_general(y, w2_ref[...], (((1,), (1,)), ((), ())),
                                 preferred_element_type=jnp.float32)


_mlp_tc = pl.pallas_call(
    _mlp_body,
    out_shape=jax.ShapeDtypeStruct((B, D), jnp.float32),
)


def kernel(h, batch, vn_h, W1, W2):
    batch = batch.astype(jnp.int32)
    partial = _seg_sum_sc(h, batch.reshape(N // 128, 128))
    vn_new = _mlp_tc(partial, vn_h, W1, W2)
    h_new = _gather_add_sc(h.reshape(N * D), batch, vn_new.reshape(B * D))
    return (h_new.reshape(N, D), vn_new)


# speedup vs baseline: 1.1342x; 1.0240x over previous
import functools

import jax
import jax.numpy as jnp
from jax import lax
from jax.experimental import pallas as pl
from jax.experimental.pallas import tpu as pltpu
from jax.experimental.pallas import tpu_sc as plsc

N = 32768
B = 16
D = 128
L = 16
DC = D // L
NC, NS = 2, 16
NW = NC * NS
RPW = N // NW
CH = 256
G = RPW // CH
KI = 128
_MESH = plsc.VectorSubcoreMesh(core_axis_name="c", subcore_axis_name="s")


@functools.partial(
    pl.kernel,
    out_type=jax.ShapeDtypeStruct((NC, B, D), jnp.float32),
    mesh=_MESH,
    scratch_types=[
        pltpu.VMEM((RPW // KI, KI), jnp.int32),
        pltpu.VMEM((2, CH, D), jnp.float32),
        pltpu.VMEM((B, D), jnp.float32),
        pltpu.VMEM_SHARED((B, D), jnp.float32),
        pltpu.SemaphoreType.DMA,
        pltpu.SemaphoreType.DMA,
        pltpu.SemaphoreType.DMA,
        pltpu.SemaphoreType.DMA,
    ],
)
def _seg_sum_sc(h_hbm, batch_hbm, out_hbm, bidx, buf, zbuf, acc_sh,
                sem0, sem1, ssem0, ssem1):
    cid = lax.axis_index("c")
    sid = lax.axis_index("s")
    wid = sid * NC + cid
    row0 = wid * RPW
    pltpu.sync_copy(batch_hbm.at[pl.ds(wid * (RPW // KI), RPW // KI)], bidx)

    @pl.when(sid == 0)
    def _():
        zv = jnp.zeros((L,), jnp.float32)
        for i in range(B):
            for c in range(DC):
                zbuf[i, pl.ds(c * L, L)] = zv
        pltpu.sync_copy(zbuf, acc_sh)

    plsc.subcore_barrier()
    sems = (sem0, sem1)
    ssems = (ssem0, ssem1)
    cps = [pltpu.async_copy(h_hbm.at[pl.ds(row0, CH)], buf.at[0], sems[0]),
           None]
    scps = [None, None]
    for g in range(G):
        b = g % 2
        if scps[b] is not None:
            for cp in scps[b]:
                cp.wait()
        cps[b].wait()
        if g + 1 < G:
            if scps[1 - b] is not None:
                for cp in scps[1 - b]:
                    cp.wait()
                scps[1 - b] = None
            cps[1 - b] = pltpu.async_copy(
                h_hbm.at[pl.ds(row0 + (g + 1) * CH, CH)], buf.at[1 - b],
                sems[1 - b])
        scps[b] = [
            pltpu.async_copy(buf.at[b, pl.ds(k * KI, KI)],
                             acc_sh.at[bidx.at[g * (CH // KI) + k]],
                             ssems[b], add=True)
            for k in range(CH // KI)]
    for sc in scps:
        if sc is not None:
            for cp in sc:
                cp.wait()
    plsc.subcore_barrier()

    @pl.when(sid == 0)
    def _():
        pltpu.sync_copy(acc_sh, out_hbm.at[cid])


@functools.partial(
    pl.kernel,
    out_type=jax.ShapeDtypeStruct((N * D,), jnp.float32),
    mesh=_MESH,
    scratch_types=[
        pltpu.VMEM((RPW,), jnp.int32),
        (pltpu.VMEM((CH * D,), jnp.float32),
         pltpu.VMEM((CH * D,), jnp.float32),
         pltpu.VMEM((CH * D,), jnp.float32)),
        pltpu.VMEM((B * D,), jnp.float32),
        (pltpu.SemaphoreType.DMA, pltpu.SemaphoreType.DMA,
         pltpu.SemaphoreType.DMA),
        (pltpu.SemaphoreType.DMA, pltpu.SemaphoreType.DMA,
         pltpu.SemaphoreType.DMA),
    ],
)
def _gather_add_sc(h_hbm, batch_hbm, vn_hbm, out_hbm, batch_v, buf, vn_v,
                   isems, osems):
    wid = lax.axis_index("s") * NC + lax.axis_index("c")
    row0 = wid * RPW
    pltpu.sync_copy(batch_hbm.at[pl.ds(row0, RPW)], batch_v)
    pltpu.sync_copy(vn_hbm, vn_v)
    icps = [None, None, None]
    ocps = [None, None, None]
    for m in range(min(2, G)):
        icps[m] = pltpu.async_copy(
            h_hbm.at[pl.ds((row0 + m * CH) * D, CH * D)], buf[m],
            isems[m])
    for g in range(G):
        m = g % 3
        if g >= 2:
            ocps[(g - 2) % 3].wait()
        if g + 1 < G and g >= 1:
            icps[(g + 1) % 3] = pltpu.async_copy(
                h_hbm.at[pl.ds((row0 + (g + 1) * CH) * D, CH * D)],
                buf[(g + 1) % 3], isems[(g + 1) % 3])
        icps[m].wait()
        s_first = batch_v[pl.ds(g * CH, L)][0]
        s_last = batch_v[pl.ds(g * CH + CH - L, L)][L - 1]

        def fast(g=g, m=m, s=s_first):
            vnrow = [vn_v[pl.ds(s * D + c * L, L)] for c in range(DC)]

            def fgrp(t, carry):
                for j in range(L):
                    for c in range(DC):
                        plsc.addupdate(
                            buf[m].at[pl.ds(t * (L * D) + j * D + c * L, L)],
                            vnrow[c])
                return carry

            lax.fori_loop(0, CH // L, fgrp, 0)

        def slow(g=g, m=m):
            def grp_body(t, carry):
                segv = batch_v[pl.ds(g * CH + t * L, L)]
                for j in range(L):
                    o = segv[j] * D
                    for c in range(DC):
                        plsc.addupdate(
                            buf[m].at[pl.ds(t * (L * D) + j * D + c * L, L)],
                            vn_v[pl.ds(o + c * L, L)])
                return carry

            lax.fori_loop(0, CH // L, grp_body, 0)

        lax.cond(s_first == s_last, fast, slow)

        ocps[m] = pltpu.async_copy(
            buf[m], out_hbm.at[pl.ds((row0 + g * CH) * D, CH * D)],
            osems[m])
    for g in range(max(0, G - 2), G):
        ocps[g % 3].wait()


def _mlp_body(p_ref, vn_ref, w1_ref, w2_ref, o_ref):
    x = jnp.sum(p_ref[...], axis=0) + vn_ref[...]
    y = jnp.maximum(
        lax.dot_general(x, w1_ref[...], (((1,), (1,)), ((), ())),
                        preferred_element_type=jnp.float32), 0.0)
    o_ref[...] = lax.dot_general(y, w2_ref[...], (((1,), (1,)), ((), ())),
                                 preferred_element_type=jnp.float32)


_mlp_tc = pl.pallas_call(
    _mlp_body,
    out_shape=jax.ShapeDtypeStruct((B, D), jnp.float32),
)


def kernel(h, batch, vn_h, W1, W2):
    batch = batch.astype(jnp.int32)
    partial = _seg_sum_sc(h, batch.reshape(N // 128, 128))
    vn_new = _mlp_tc(partial, vn_h, W1, W2)
    h_new = _gather_add_sc(h.reshape(N * D), batch, vn_new.reshape(B * D))
    return (h_new.reshape(N, D), vn_new)
